# baseline (device time: 19825 ns/iter reference)
import functools

import jax
import jax.numpy as jnp
from jax import lax
from jax.experimental import pallas as pl
from jax.experimental.pallas import tpu as pltpu

N_PAIRS = 8


def _exchange_add(raw, local_ids_2d, v_shard):
    t, d = raw.shape

    def body(
        raw_ref, loc_ref, out_ref,
        send_buf, recv_buf, send_sem, recv_sems, ack_sems,
    ):
        my_x = lax.axis_index("x")
        my_y = lax.axis_index("y")
        my_z = lax.axis_index("z")
        nbr = (1 - my_x, my_y, my_z)
        p = my_y * 4 + my_z

        loc = loc_ref[...]
        mask = (loc >= 0) & (loc < v_shard)
        send_buf[...] = jnp.where(mask, raw_ref[...], 0.0).astype(jnp.bfloat16)

        rdma = pltpu.make_async_remote_copy(
            src_ref=send_buf,
            dst_ref=recv_buf.at[p],
            send_sem=send_sem,
            recv_sem=recv_sems.at[p],
            device_id=nbr,
            device_id_type=pl.DeviceIdType.MESH,
        )
        rdma.start()
        rdma.wait()

        pl.semaphore_signal(
            ack_sems.at[p], inc=1, device_id=nbr,
            device_id_type=pl.DeviceIdType.MESH,
        )
        pl.semaphore_wait(ack_sems.at[p], 1)

        out_ref[...] = send_buf[...].astype(jnp.float32) + recv_buf[
            p
        ].astype(jnp.float32)

    return pl.pallas_call(
        body,
        out_shape=jax.ShapeDtypeStruct((t, d), jnp.float32),
        in_specs=[
            pl.BlockSpec(memory_space=pltpu.VMEM),
            pl.BlockSpec(memory_space=pltpu.VMEM),
        ],
        out_specs=pl.BlockSpec(memory_space=pltpu.VMEM),
        scratch_shapes=[
            pltpu.VMEM((t, d), jnp.bfloat16),
            pltpu.VMEM((N_PAIRS, t, d), jnp.bfloat16),
            pltpu.SemaphoreType.DMA,
            pltpu.SemaphoreType.DMA((N_PAIRS,)),
            pltpu.SemaphoreType.REGULAR((N_PAIRS,)),
        ],
    )(raw, local_ids_2d)


def kernel(ids, E):
    v_shard = E.shape[0]
    my_x = lax.axis_index("x")
    local = ids - my_x * v_shard
    raw = E[jnp.clip(local, 0, v_shard - 1)]
    return _exchange_add(raw, local[:, None], v_shard)


# device time: 14490 ns/iter; 1.3682x vs baseline; 1.3682x over previous
import functools

import jax
import jax.numpy as jnp
from jax import lax
from jax.experimental import pallas as pl
from jax.experimental.pallas import tpu as pltpu

N_PAIRS = 8


def _exchange_add(raw, local_ids_2d, v_shard):
    t, d = raw.shape

    def body(
        raw_ref, loc_ref, out_ref,
        send_buf, recv_buf, send_sem, recv_sems, ack_sems, entry_sems,
    ):
        my_x = lax.axis_index("x")
        my_y = lax.axis_index("y")
        my_z = lax.axis_index("z")
        nbr = (1 - my_x, my_y, my_z)
        p = my_y * 4 + my_z

        barrier_sem = pltpu.get_barrier_semaphore()
        pl.semaphore_signal(barrier_sem, inc=1)
        pl.semaphore_wait(barrier_sem, 1)

        loc = loc_ref[...]
        mask = (loc >= 0) & (loc < v_shard)
        send_buf[...] = jnp.where(mask, raw_ref[...], 0.0).astype(jnp.bfloat16)

        pl.semaphore_signal(
            entry_sems.at[p], inc=1, device_id=nbr,
            device_id_type=pl.DeviceIdType.MESH,
        )
        pl.semaphore_wait(entry_sems.at[p], 1)

        rdma = pltpu.make_async_remote_copy(
            src_ref=send_buf,
            dst_ref=recv_buf.at[p],
            send_sem=send_sem,
            recv_sem=recv_sems.at[p],
            device_id=nbr,
            device_id_type=pl.DeviceIdType.MESH,
        )
        rdma.start()
        rdma.wait()

        pl.semaphore_signal(
            ack_sems.at[p], inc=1, device_id=nbr,
            device_id_type=pl.DeviceIdType.MESH,
        )
        pl.semaphore_wait(ack_sems.at[p], 1)

        out_ref[...] = send_buf[...].astype(jnp.float32) + recv_buf[
            p
        ].astype(jnp.float32)

    return pl.pallas_call(
        body,
        out_shape=jax.ShapeDtypeStruct((t, d), jnp.float32),
        in_specs=[
            pl.BlockSpec(memory_space=pltpu.VMEM),
            pl.BlockSpec(memory_space=pltpu.VMEM),
        ],
        out_specs=pl.BlockSpec(memory_space=pltpu.VMEM),
        scratch_shapes=[
            pltpu.VMEM((t, d), jnp.bfloat16),
            pltpu.VMEM((N_PAIRS, t, d), jnp.bfloat16),
            pltpu.SemaphoreType.DMA,
            pltpu.SemaphoreType.DMA((N_PAIRS,)),
            pltpu.SemaphoreType.REGULAR((N_PAIRS,)),
            pltpu.SemaphoreType.REGULAR((N_PAIRS,)),
        ],
        compiler_params=pltpu.CompilerParams(collective_id=0),
    )(raw, local_ids_2d)


def kernel(ids, E):
    v_shard = E.shape[0]
    my_x = lax.axis_index("x")
    local = ids - my_x * v_shard
    raw = E[jnp.clip(local, 0, v_shard - 1)]
    return _exchange_add(raw, local[:, None], v_shard)
